# baseline (speedup 1.0000x reference)
import functools

import jax
import jax.numpy as jnp
from jax import lax
from jax.experimental import pallas as pl
from jax.experimental.pallas import tpu as pltpu
from jax.experimental.pallas import tpu_sc as plsc

VOCAB_ROWS = 1000000
D_MODEL = 64
D_WORDS = D_MODEL // 2
LANES = 16
NUM_WORKERS = 32
CHUNK = 1024

_MAGIC = 2.0 ** 112


def _f16_bits_to_f32(h):
    o = ((h & 0x8000) << 16) | ((h & 0x7FFF) << 13)
    return lax.bitcast_convert_type(o, jnp.float32) * _MAGIC


def _make_kernel(n_rows):
    per_worker = n_rows // NUM_WORKERS
    n_chunks = per_worker // CHUNK
    mesh = plsc.VectorSubcoreMesh(core_axis_name="c", subcore_axis_name="s")

    @functools.partial(
        pl.kernel,
        out_type=jax.ShapeDtypeStruct((n_rows, D_MODEL), jnp.float32),
        mesh=mesh,
        compiler_params=pltpu.CompilerParams(
            needs_layout_passes=False, use_tc_tiling_on_sc=False
        ),
        scratch_types=[
            pltpu.VMEM((CHUNK,), jnp.int32),
            pltpu.VMEM((CHUNK, D_WORDS), jnp.int32),
            pltpu.VMEM((CHUNK, D_MODEL), jnp.float32),
            pltpu.SemaphoreType.DMA,
        ],
    )
    def emb_kernel(idx_hbm, table_hbm, out_hbm, idx_v, in_v, out_v, sem):
        wid = lax.axis_index("s") * 2 + lax.axis_index("c")
        lane = lax.iota(jnp.int32, LANES)
        col_e0 = 2 * lane
        col_o0 = 2 * lane + 1
        col_e1 = col_e0 + 2 * LANES
        col_o1 = col_o0 + 2 * LANES

        def chunk_body(c, carry):
            base = wid * per_worker + c * CHUNK
            pltpu.sync_copy(idx_hbm.at[pl.ds(base, CHUNK)], idx_v)
            pltpu.async_copy(table_hbm.at[idx_v], in_v, sem).wait()

            def row_body(r, rcarry):
                rsplat = jnp.full((LANES,), 0, jnp.int32) + r
                for k, (ce, co) in enumerate(((col_e0, col_o0), (col_e1, col_o1))):
                    w = in_v[r, pl.ds(k * LANES, LANES)]
                    f_lo = _f16_bits_to_f32(w)
                    f_hi = _f16_bits_to_f32(lax.shift_right_logical(w, 16))
                    plsc.store_scatter(out_v, [rsplat, ce], f_lo)
                    plsc.store_scatter(out_v, [rsplat, co], f_hi)
                return rcarry

            lax.fori_loop(0, CHUNK, row_body, 0)
            pltpu.sync_copy(out_v, out_hbm.at[pl.ds(base, CHUNK)])
            return carry

        lax.fori_loop(0, n_chunks, chunk_body, 0)

    return emb_kernel


def kernel(token_ids, weight):
    n_rows = token_ids.size
    idx = token_ids.reshape(-1).astype(jnp.int32)
    table_i32 = lax.bitcast_convert_type(
        weight.reshape(VOCAB_ROWS, D_WORDS, 2), jnp.int32
    )
    out = _make_kernel(n_rows)(idx, table_i32)
    return out.reshape(*token_ids.shape, D_MODEL)


# speedup vs baseline: 1.0363x; 1.0363x over previous
import functools

import jax
import jax.numpy as jnp
from jax import lax
from jax.experimental import pallas as pl
from jax.experimental.pallas import tpu as pltpu
from jax.experimental.pallas import tpu_sc as plsc

VOCAB_ROWS = 1000000
D_MODEL = 64
D_WORDS = D_MODEL // 2
LANES = 16
NUM_WORKERS = 32
CHUNK = 512
ROW_UNROLL = 4

_MAGIC = 2.0 ** 112


def _f16_bits_to_f32(h):
    o = ((h & 0x8000) << 16) | ((h & 0x7FFF) << 13)
    return lax.bitcast_convert_type(o, jnp.float32) * _MAGIC


def _make_kernel(n_rows):
    per_worker = n_rows // NUM_WORKERS
    n_chunks = per_worker // CHUNK
    assert n_chunks % 2 == 0
    mesh = plsc.VectorSubcoreMesh(core_axis_name="c", subcore_axis_name="s")

    @functools.partial(
        pl.kernel,
        out_type=jax.ShapeDtypeStruct((n_rows, D_MODEL), jnp.float32),
        mesh=mesh,
        compiler_params=pltpu.CompilerParams(
            needs_layout_passes=False, use_tc_tiling_on_sc=False
        ),
        scratch_types=[
            pltpu.VMEM((2, CHUNK), jnp.int32),
            pltpu.VMEM((2, CHUNK, D_WORDS), jnp.int32),
            pltpu.VMEM((2, CHUNK, D_MODEL), jnp.float32),
            pltpu.SemaphoreType.DMA,
            pltpu.SemaphoreType.DMA,
            pltpu.SemaphoreType.DMA,
            pltpu.SemaphoreType.DMA,
        ],
    )
    def emb_kernel(idx_hbm, table_hbm, out_hbm, idx_v, in_v, out_v,
                   gsem0, gsem1, osem0, osem1):
        wid = lax.axis_index("s") * 2 + lax.axis_index("c")
        w_base = wid * per_worker
        lane = lax.iota(jnp.int32, LANES)
        cols = []
        for k in range(2):
            cols.append((2 * lane + 2 * LANES * k, 2 * lane + 2 * LANES * k + 1))
        gsems = (gsem0, gsem1)
        osems = (osem0, osem1)

        def start_gather(c, b):
            base = w_base + c * CHUNK
            pltpu.sync_copy(idx_hbm.at[pl.ds(base, CHUNK)], idx_v.at[b])
            pltpu.async_copy(table_hbm.at[idx_v.at[b]], in_v.at[b], gsems[b])

        def convert(b):
            def row_body(r0, carry):
                for dr in range(ROW_UNROLL):
                    r = r0 * ROW_UNROLL + dr
                    rsplat = jnp.zeros((LANES,), jnp.int32) + r
                    for k, (ce, co) in enumerate(cols):
                        w = in_v[b, r, pl.ds(k * LANES, LANES)]
                        f_lo = _f16_bits_to_f32(w)
                        f_hi = _f16_bits_to_f32(lax.shift_right_logical(w, 16))
                        plsc.store_scatter(out_v.at[b], [rsplat, ce], f_lo)
                        plsc.store_scatter(out_v.at[b], [rsplat, co], f_hi)
                return carry

            lax.fori_loop(0, CHUNK // ROW_UNROLL, row_body, 0)

        start_gather(0, 0)

        def outer(c2, carry):
            for b in range(2):
                c = 2 * c2 + b
                @pl.when(c + 1 < n_chunks)
                def _():
                    start_gather(c + 1, 1 - b)

                pltpu.make_async_copy(
                    table_hbm.at[idx_v.at[b]], in_v.at[b], gsems[b]
                ).wait()
                @pl.when(c >= 2)
                def _():
                    pltpu.make_async_copy(
                        out_v.at[b], out_hbm.at[pl.ds(w_base, CHUNK)], osems[b]
                    ).wait()

                convert(b)
                base = w_base + c * CHUNK
                pltpu.async_copy(
                    out_v.at[b], out_hbm.at[pl.ds(base, CHUNK)], osems[b]
                )
            return carry

        lax.fori_loop(0, n_chunks // 2, outer, 0)

        for b in range(2):
            pltpu.make_async_copy(
                out_v.at[b], out_hbm.at[pl.ds(w_base, CHUNK)], osems[b]
            ).wait()

    return emb_kernel


def kernel(token_ids, weight):
    n_rows = token_ids.size
    idx = token_ids.reshape(-1).astype(jnp.int32)
    table_i32 = lax.bitcast_convert_type(
        weight.reshape(VOCAB_ROWS, D_WORDS, 2), jnp.int32
    )
    out = _make_kernel(n_rows)(idx, table_i32)
    return out.reshape(*token_ids.shape, D_MODEL)


# speedup vs baseline: 1.6074x; 1.5511x over previous
import functools

import jax
import jax.numpy as jnp
from jax import lax
from jax.experimental import pallas as pl
from jax.experimental.pallas import tpu as pltpu
from jax.experimental.pallas import tpu_sc as plsc

D_MODEL = 64
LANES = 16
NUM_WORKERS = 32
CHUNK_TROWS = 8

_MAGIC = 2.0 ** 112


def _f16_bits_to_f32(h):
    o = ((h & 0x8000) << 16) | ((h & 0x7FFF) << 13)
    return lax.bitcast_convert_type(o, jnp.float32) * _MAGIC


def _make_kernel(n_tokens, seq):
    trows_per_worker = n_tokens // NUM_WORKERS
    n_chunks = trows_per_worker // CHUNK_TROWS
    assert n_chunks % 2 == 0
    chunk_rows = CHUNK_TROWS * seq
    mesh = plsc.VectorSubcoreMesh(core_axis_name="c", subcore_axis_name="s")

    @functools.partial(
        pl.kernel,
        out_type=jax.ShapeDtypeStruct((n_tokens, seq, D_MODEL), jnp.float32),
        mesh=mesh,
        compiler_params=pltpu.CompilerParams(
            needs_layout_passes=False, use_tc_tiling_on_sc=False
        ),
        scratch_types=[
            pltpu.VMEM((2, chunk_rows), jnp.int32),
            pltpu.VMEM((2, chunk_rows, D_MODEL), jnp.float16),
            pltpu.VMEM((2, CHUNK_TROWS, seq, D_MODEL), jnp.float32),
            pltpu.SemaphoreType.DMA,
            pltpu.SemaphoreType.DMA,
            pltpu.SemaphoreType.DMA,
            pltpu.SemaphoreType.DMA,
        ],
    )
    def emb_kernel(idx_hbm, table_hbm, out_hbm, idx_v, in_v, out_v,
                   gsem0, gsem1, osem0, osem1):
        wid = lax.axis_index("s") * 2 + lax.axis_index("c")
        w_row_base = wid * trows_per_worker
        lane = lax.iota(jnp.int32, LANES)
        zeros = jnp.zeros((LANES,), jnp.int32)
        cols = []
        for k in range(2):
            cols.append((2 * lane + 2 * LANES * k, 2 * lane + 2 * LANES * k + 1))
        gsems = (gsem0, gsem1)
        osems = (osem0, osem1)

        def start_gather(c, b):
            base = (w_row_base + c * CHUNK_TROWS) * seq
            pltpu.sync_copy(idx_hbm.at[pl.ds(base, chunk_rows)], idx_v.at[b])
            pltpu.async_copy(table_hbm.at[idx_v.at[b]], in_v.at[b], gsems[b])

        def convert(b):
            def trow_body(tr, carry):
                trs = zeros + tr

                def tok_body(t, tcarry):
                    ts = zeros + t
                    e = tr * seq + t
                    for k, (ce, co) in enumerate(cols):
                        w16 = in_v[b, e, pl.ds(k * 2 * LANES, 2 * LANES)]
                        w = plsc.bitcast(w16, jnp.int32)
                        f_lo = _f16_bits_to_f32(w)
                        f_hi = _f16_bits_to_f32(lax.shift_right_logical(w, 16))
                        plsc.store_scatter(out_v.at[b], [trs, ts, ce], f_lo)
                        plsc.store_scatter(out_v.at[b], [trs, ts, co], f_hi)
                    return tcarry

                lax.fori_loop(0, seq, tok_body, 0)
                return carry

            lax.fori_loop(0, CHUNK_TROWS, trow_body, 0)

        start_gather(0, 0)

        def outer(c2, carry):
            for b in range(2):
                c = 2 * c2 + b
                @pl.when(c + 1 < n_chunks)
                def _():
                    start_gather(c + 1, 1 - b)

                pltpu.make_async_copy(
                    table_hbm.at[idx_v.at[b]], in_v.at[b], gsems[b]
                ).wait()
                @pl.when(c >= 2)
                def _():
                    pltpu.make_async_copy(
                        out_v.at[b],
                        out_hbm.at[pl.ds(w_row_base, CHUNK_TROWS)],
                        osems[b],
                    ).wait()

                convert(b)
                row_base = w_row_base + c * CHUNK_TROWS
                pltpu.async_copy(
                    out_v.at[b], out_hbm.at[pl.ds(row_base, CHUNK_TROWS)], osems[b]
                )
            return carry

        lax.fori_loop(0, n_chunks // 2, outer, 0)

        for b in range(2):
            pltpu.make_async_copy(
                out_v.at[b], out_hbm.at[pl.ds(w_row_base, CHUNK_TROWS)], osems[b]
            ).wait()

    return emb_kernel


def kernel(token_ids, weight):
    n_tokens, seq = token_ids.shape
    idx = token_ids.reshape(-1).astype(jnp.int32)
    out = _make_kernel(n_tokens, seq)(idx, weight)
    return out


# speedup vs baseline: 1.6541x; 1.0290x over previous
import functools

import jax
import jax.numpy as jnp
from jax import lax
from jax.experimental import pallas as pl
from jax.experimental.pallas import tpu as pltpu
from jax.experimental.pallas import tpu_sc as plsc

D_MODEL = 64
LANES = 16
NUM_WORKERS = 32
CHUNK_TROWS = 8
TOK_UNROLL = 5

_MAGIC = 2.0 ** 112


def _f16_bits_to_f32(h):
    o = ((h & 0x8000) << 16) | ((h & 0x7FFF) << 13)
    return lax.bitcast_convert_type(o, jnp.float32) * _MAGIC


def _make_kernel(n_tokens, seq):
    trows_per_worker = n_tokens // NUM_WORKERS
    n_chunks = trows_per_worker // CHUNK_TROWS
    assert n_chunks % 2 == 0
    chunk_rows = CHUNK_TROWS * seq
    mesh = plsc.VectorSubcoreMesh(core_axis_name="c", subcore_axis_name="s")

    @functools.partial(
        pl.kernel,
        out_type=jax.ShapeDtypeStruct((n_tokens, seq, D_MODEL), jnp.float32),
        mesh=mesh,
        compiler_params=pltpu.CompilerParams(
            needs_layout_passes=False, use_tc_tiling_on_sc=False
        ),
        scratch_types=[
            pltpu.VMEM((trows_per_worker * seq,), jnp.int32),
            pltpu.VMEM((2, chunk_rows, D_MODEL), jnp.float16),
            pltpu.VMEM((2, CHUNK_TROWS, seq, D_MODEL), jnp.float32),
            pltpu.SemaphoreType.DMA,
            pltpu.SemaphoreType.DMA,
            pltpu.SemaphoreType.DMA,
            pltpu.SemaphoreType.DMA,
        ],
    )
    def emb_kernel(idx_hbm, table_hbm, out_hbm, idx_v, in_v, out_v,
                   gsem0, gsem1, osem0, osem1):
        wid = lax.axis_index("s") * 2 + lax.axis_index("c")
        w_row_base = wid * trows_per_worker
        lane = lax.iota(jnp.int32, LANES)
        zeros = jnp.zeros((LANES,), jnp.int32)
        cols = []
        for k in range(2):
            cols.append((2 * lane + 2 * LANES * k, 2 * lane + 2 * LANES * k + 1))
        gsems = (gsem0, gsem1)
        osems = (osem0, osem1)

        def start_gather(c, b):
            idx_slice = idx_v.at[pl.ds(c * chunk_rows, chunk_rows)]
            pltpu.async_copy(table_hbm.at[idx_slice], in_v.at[b], gsems[b])

        def wait_gather(c, b):
            idx_slice = idx_v.at[pl.ds(c * chunk_rows, chunk_rows)]
            pltpu.make_async_copy(
                table_hbm.at[idx_slice], in_v.at[b], gsems[b]
            ).wait()

        def convert(b):
            def trow_body(tr, carry):
                trs = zeros + tr

                def tok_body(t2, tcarry):
                    for dt in range(TOK_UNROLL):
                        t = t2 * TOK_UNROLL + dt
                        ts = zeros + t
                        e = tr * seq + t
                        for k, (ce, co) in enumerate(cols):
                            w16 = in_v[b, e, pl.ds(k * 2 * LANES, 2 * LANES)]
                            w = plsc.bitcast(w16, jnp.int32)
                            o_lo = ((w & 0x8000) << 16) | ((w & 0x7FFF) << 13)
                            o_hi = (w & -0x80000000) | (
                                lax.shift_right_logical(w, 3) & 0x0FFFE000
                            )
                            f_lo = lax.bitcast_convert_type(o_lo, jnp.float32) * _MAGIC
                            f_hi = lax.bitcast_convert_type(o_hi, jnp.float32) * _MAGIC
                            plsc.store_scatter(out_v.at[b], [trs, ts, ce], f_lo)
                            plsc.store_scatter(out_v.at[b], [trs, ts, co], f_hi)
                    return tcarry

                lax.fori_loop(0, seq // TOK_UNROLL, tok_body, 0)
                return carry

            lax.fori_loop(0, CHUNK_TROWS, trow_body, 0)

        pltpu.sync_copy(
            idx_hbm.at[pl.ds(w_row_base * seq, trows_per_worker * seq)], idx_v
        )
        start_gather(0, 0)

        def outer(c2, carry):
            for b in range(2):
                c = 2 * c2 + b
                @pl.when(c + 1 < n_chunks)
                def _():
                    start_gather(c + 1, 1 - b)

                wait_gather(c, b)
                @pl.when(c >= 2)
                def _():
                    pltpu.make_async_copy(
                        out_v.at[b],
                        out_hbm.at[pl.ds(w_row_base, CHUNK_TROWS)],
                        osems[b],
                    ).wait()

                convert(b)
                row_base = w_row_base + c * CHUNK_TROWS
                pltpu.async_copy(
                    out_v.at[b], out_hbm.at[pl.ds(row_base, CHUNK_TROWS)], osems[b]
                )
            return carry

        lax.fori_loop(0, n_chunks // 2, outer, 0)

        for b in range(2):
            pltpu.make_async_copy(
                out_v.at[b], out_hbm.at[pl.ds(w_row_base, CHUNK_TROWS)], osems[b]
            ).wait()

    return emb_kernel


def kernel(token_ids, weight):
    n_tokens, seq = token_ids.shape
    idx = token_ids.reshape(-1).astype(jnp.int32)
    out = _make_kernel(n_tokens, seq)(idx, weight)
    return out
